# baseline (speedup 1.0000x reference)
import functools

import jax
import jax.numpy as jnp
from jax import lax
from jax.experimental import pallas as pl
from jax.experimental.pallas import tpu as pltpu
from jax.experimental.pallas import tpu_sc as plsc

FEATURE_DIM = 128
BOARD_SIZE = 15
PCODE_DIM = 2380
EMBED_DIM = 2 * (PCODE_DIM + 1)
CELL_DIM = BOARD_SIZE * BOARD_SIZE

NUM_CORES = 2
NUM_SUBCORES = 16
LANES = 16

STRIPE = 304
REG_ROWS = 4768
CELLS0 = 112
SG = 32


def _sc_gather_sum(idx_in, w_pcode, w_board, batch):
    bpt = batch // NUM_SUBCORES
    rec = 4 * bpt
    n_board = CELL_DIM * EMBED_DIM
    mesh = plsc.VectorSubcoreMesh(core_axis_name="c", subcore_axis_name="s")

    @functools.partial(
        pl.kernel,
        mesh=mesh,
        out_type=jax.ShapeDtypeStruct((CELL_DIM, batch, FEATURE_DIM),
                                      jnp.float32),
        scratch_types=[
            pltpu.VMEM_SHARED((EMBED_DIM, FEATURE_DIM), jnp.float32),
            pltpu.VMEM_SHARED((2, REG_ROWS, FEATURE_DIM), jnp.float32),
            pltpu.VMEM((rec,), jnp.int32),
            pltpu.VMEM((4, SG, FEATURE_DIM), jnp.float32),
            pltpu.SemaphoreType.DMA,
            pltpu.SemaphoreType.DMA,
            pltpu.SemaphoreType.DMA,
        ],
    )
    def k(idx_hbm, wp_hbm, wb_hbm, out_hbm, wp_sp, reg_sp, raw_v,
          g_v, gsem, osem, ssem):
        cid = lax.axis_index("c")
        sid = lax.axis_index("s")

        @pl.when(sid < NUM_SUBCORES - 1)
        def _():
            off = pl.multiple_of(sid * STRIPE, 8)
            pltpu.sync_copy(wp_hbm.at[pl.ds(off, STRIPE)],
                            wp_sp.at[pl.ds(off, STRIPE)])

        @pl.when(sid == NUM_SUBCORES - 1)
        def _():
            tail = EMBED_DIM - 15 * STRIPE
            pltpu.sync_copy(wp_hbm.at[pl.ds(15 * STRIPE, tail)],
                            wp_sp.at[pl.ds(15 * STRIPE, tail)])

        c_lo = cid * CELLS0
        n_cells = CELLS0 + cid

        def reg_start(c):
            src0 = c * EMBED_DIM
            start = pl.multiple_of(src0 // 8 * 8, 8)
            return start, src0 - start

        def stage_region(c, buf):
            start, _ = reg_start(c)

            @pl.when(sid < NUM_SUBCORES - 1)
            def _():
                off = pl.multiple_of(sid * STRIPE, 8)
                pltpu.async_copy(wb_hbm.at[pl.ds(start + off, STRIPE)],
                                 reg_sp.at[buf, pl.ds(off, STRIPE)], ssem)

            @pl.when(sid == NUM_SUBCORES - 1)
            def _():
                off = 15 * STRIPE
                @pl.when(start + REG_ROWS <= n_board)
                def _():
                    pltpu.async_copy(
                        wb_hbm.at[pl.ds(start + off, REG_ROWS - off)],
                        reg_sp.at[buf, pl.ds(off, REG_ROWS - off)], ssem)

                @pl.when(start + REG_ROWS > n_board)
                def _():
                    pltpu.async_copy(
                        wb_hbm.at[pl.ds(start + off, 192)],
                        reg_sp.at[buf, pl.ds(off, 192)], ssem)

                    @pl.loop(0, 10)
                    def _(i):
                        pltpu.sync_copy(
                            wb_hbm.at[pl.ds(start + off + 192 + i, 1)],
                            reg_sp.at[buf, pl.ds(off + 192 + i, 1)])

        def wait_region(c, buf):
            start, _ = reg_start(c)

            @pl.when(sid < NUM_SUBCORES - 1)
            def _():
                pltpu.make_async_copy(
                    wb_hbm.at[pl.ds(0, STRIPE)],
                    reg_sp.at[buf, pl.ds(0, STRIPE)], ssem).wait()

            @pl.when(sid == NUM_SUBCORES - 1)
            def _():
                off = 15 * STRIPE
                @pl.when(start + REG_ROWS <= n_board)
                def _():
                    pltpu.make_async_copy(
                        wb_hbm.at[pl.ds(0, REG_ROWS - off)],
                        reg_sp.at[buf, pl.ds(0, REG_ROWS - off)], ssem).wait()

                @pl.when(start + REG_ROWS > n_board)
                def _():
                    pltpu.make_async_copy(
                        wb_hbm.at[pl.ds(0, 192)],
                        reg_sp.at[buf, pl.ds(0, 192)], ssem).wait()

        def compute_cell(c, buf, next_c, next_buf, do_stage):
            _, skew = reg_start(c)
            reg = reg_sp.at[buf]

            pltpu.sync_copy(
                idx_hbm.at[pl.ds((c * NUM_SUBCORES + sid) * rec, rec)],
                raw_v)

            @pl.loop(0, bpt // LANES)
            def _(j):
                sl0 = pl.ds(j * LANES, LANES)
                sl1 = pl.ds(bpt + j * LANES, LANES)
                occ = (raw_v[pl.ds(2 * bpt + j * LANES, LANES)]
                       + raw_v[pl.ds(3 * bpt + j * LANES, LANES)]) > 0
                p0 = jnp.where(occ, PCODE_DIM, raw_v[sl0])
                p1 = jnp.where(occ, 2 * PCODE_DIM + 1,
                               raw_v[sl1] + (PCODE_DIM + 1))
                raw_v[sl0] = p0
                raw_v[sl1] = p1
                raw_v[pl.ds(2 * bpt + j * LANES, LANES)] = p0 + skew
                raw_v[pl.ds(3 * bpt + j * LANES, LANES)] = p1 + skew

            for g in range(bpt // SG):
                if g == 1 and do_stage is not None:
                    @pl.when(do_stage)
                    def _():
                        stage_region(next_c, next_buf)
                i0 = g * SG
                s0 = raw_v.at[pl.ds(i0, SG)]
                s1 = raw_v.at[pl.ds(bpt + i0, SG)]
                s2 = raw_v.at[pl.ds(2 * bpt + i0, SG)]
                s3 = raw_v.at[pl.ds(3 * bpt + i0, SG)]
                pltpu.async_copy(wp_sp.at[s0], g_v.at[0], gsem)
                pltpu.async_copy(wp_sp.at[s1], g_v.at[1], gsem)
                pltpu.async_copy(reg.at[s2], g_v.at[2], gsem)
                pltpu.async_copy(reg.at[s3], g_v.at[3], gsem)
                for j, tab in enumerate((wp_sp.at[s0], wp_sp.at[s1],
                                         reg.at[s2], reg.at[s3])):
                    pltpu.make_async_copy(tab, g_v.at[j], gsem).wait()

                @pl.loop(0, SG)
                def _(r):
                    for q in range(FEATURE_DIM // LANES):
                        sl = pl.ds(q * LANES, LANES)
                        g_v[3, r, sl] = (g_v[0, r, sl] + g_v[1, r, sl]) + (
                            g_v[2, r, sl] + g_v[3, r, sl])

                boff = pl.multiple_of(sid * bpt + i0, 8)
                pltpu.async_copy(
                    g_v.at[3], out_hbm.at[c, pl.ds(boff, SG)], osem)

            for g in range(bpt // SG):
                boff = pl.multiple_of(sid * bpt + g * SG, 8)
                pltpu.make_async_copy(
                    g_v.at[3], out_hbm.at[c, pl.ds(boff, SG)], osem).wait()

        plsc.subcore_barrier()

        stage_region(c_lo, 0)
        wait_region(c_lo, 0)
        plsc.subcore_barrier()

        n_pairs = n_cells // 2

        def do_pair(t, _):
            a = c_lo + 2 * t
            compute_cell(a, 0, a + 1, 1, do_stage=a + 1 < c_lo + n_cells)
            wait_region(a + 1, 1)
            plsc.subcore_barrier()
            more = a + 2 < c_lo + n_cells
            compute_cell(a + 1, 1, a + 2, 0, do_stage=more)

            @pl.when(more)
            def _():
                wait_region(a + 2, 0)

            plsc.subcore_barrier()
            return 0

        lax.fori_loop(0, n_pairs, do_pair, 0)

        @pl.when(n_cells % 2 == 1)
        def _():
            compute_cell(c_lo + n_cells - 1, 0, 0, 1, do_stage=None)

    return k(idx_in, w_pcode, w_board)


def _tc_transpose(s, batch):
    grp = 8

    def body(s_ref, o_ref):
        o_ref[...] = jnp.transpose(s_ref[...], (1, 2, 0))

    return pl.pallas_call(
        body,
        grid=(batch // grp,),
        in_specs=[
            pl.BlockSpec((CELL_DIM, grp, FEATURE_DIM), lambda i: (0, i, 0))
        ],
        out_specs=pl.BlockSpec((grp, FEATURE_DIM, CELL_DIM), lambda i: (i, 0, 0)),
        out_shape=jax.ShapeDtypeStruct((batch, FEATURE_DIM, CELL_DIM), jnp.float32),
    )(s)


def kernel(sparse_feature_input, sparse_feature_dim, board_input, W_pcode, W_board):
    del sparse_feature_dim
    batch = sparse_feature_input.shape[0]
    bpt = batch // NUM_SUBCORES
    pc = sparse_feature_input[:, 10:12].reshape(batch, 2, CELL_DIM)
    bd = board_input.reshape(batch, 2, CELL_DIM)
    raw = jnp.concatenate([pc, bd], axis=1)
    idx_in = jnp.transpose(raw, (2, 1, 0)).reshape(
        CELL_DIM, 4, NUM_SUBCORES, bpt)
    idx_in = jnp.transpose(idx_in, (0, 2, 1, 3)).reshape(-1)
    s = _sc_gather_sum(idx_in, W_pcode, W_board, batch)
    out = _tc_transpose(s, batch)
    return out.reshape(batch, FEATURE_DIM, BOARD_SIZE, BOARD_SIZE)


# speedup vs baseline: 1.0421x; 1.0421x over previous
import functools

import jax
import jax.numpy as jnp
from jax import lax
from jax.experimental import pallas as pl
from jax.experimental.pallas import tpu as pltpu
from jax.experimental.pallas import tpu_sc as plsc

FEATURE_DIM = 128
BOARD_SIZE = 15
PCODE_DIM = 2380
EMBED_DIM = 2 * (PCODE_DIM + 1)
CELL_DIM = BOARD_SIZE * BOARD_SIZE

NUM_CORES = 2
NUM_SUBCORES = 16
LANES = 16

STRIPE = 304
REG_ROWS = 4768
CELLS0 = 112
SG = 16


def _sc_gather_sum(idx_in, w_pcode, w_board, batch):
    bpt = batch // NUM_SUBCORES
    rec = 4 * bpt
    n_board = CELL_DIM * EMBED_DIM
    mesh = plsc.VectorSubcoreMesh(core_axis_name="c", subcore_axis_name="s")

    @functools.partial(
        pl.kernel,
        mesh=mesh,
        out_type=jax.ShapeDtypeStruct((CELL_DIM, batch, FEATURE_DIM),
                                      jnp.float32),
        scratch_types=[
            pltpu.VMEM_SHARED((EMBED_DIM, FEATURE_DIM), jnp.float32),
            pltpu.VMEM_SHARED((2, REG_ROWS, FEATURE_DIM), jnp.float32),
            pltpu.VMEM((rec,), jnp.int32),
            pltpu.VMEM((4, SG, FEATURE_DIM), jnp.float32),
            pltpu.SemaphoreType.DMA,
            pltpu.SemaphoreType.DMA,
            pltpu.SemaphoreType.DMA,
        ],
    )
    def k(idx_hbm, wp_hbm, wb_hbm, out_hbm, wp_sp, reg_sp, raw_v,
          g_v, gsem, osem, ssem):
        cid = lax.axis_index("c")
        sid = lax.axis_index("s")

        @pl.when(sid < NUM_SUBCORES - 1)
        def _():
            off = pl.multiple_of(sid * STRIPE, 8)
            pltpu.sync_copy(wp_hbm.at[pl.ds(off, STRIPE)],
                            wp_sp.at[pl.ds(off, STRIPE)])

        @pl.when(sid == NUM_SUBCORES - 1)
        def _():
            tail = EMBED_DIM - 15 * STRIPE
            pltpu.sync_copy(wp_hbm.at[pl.ds(15 * STRIPE, tail)],
                            wp_sp.at[pl.ds(15 * STRIPE, tail)])

        c_lo = cid * CELLS0
        n_cells = CELLS0 + cid

        def reg_start(c):
            src0 = c * EMBED_DIM
            start = pl.multiple_of(src0 // 8 * 8, 8)
            return start, src0 - start

        def stage_region(c, buf):
            start, _ = reg_start(c)

            @pl.when(sid < NUM_SUBCORES - 1)
            def _():
                off = pl.multiple_of(sid * STRIPE, 8)
                pltpu.async_copy(wb_hbm.at[pl.ds(start + off, STRIPE)],
                                 reg_sp.at[buf, pl.ds(off, STRIPE)], ssem)

            @pl.when(sid == NUM_SUBCORES - 1)
            def _():
                off = 15 * STRIPE
                @pl.when(start + REG_ROWS <= n_board)
                def _():
                    pltpu.async_copy(
                        wb_hbm.at[pl.ds(start + off, REG_ROWS - off)],
                        reg_sp.at[buf, pl.ds(off, REG_ROWS - off)], ssem)

                @pl.when(start + REG_ROWS > n_board)
                def _():
                    pltpu.async_copy(
                        wb_hbm.at[pl.ds(start + off, 192)],
                        reg_sp.at[buf, pl.ds(off, 192)], ssem)

                    @pl.loop(0, 10)
                    def _(i):
                        pltpu.sync_copy(
                            wb_hbm.at[pl.ds(start + off + 192 + i, 1)],
                            reg_sp.at[buf, pl.ds(off + 192 + i, 1)])

        def wait_region(c, buf):
            start, _ = reg_start(c)

            @pl.when(sid < NUM_SUBCORES - 1)
            def _():
                pltpu.make_async_copy(
                    wb_hbm.at[pl.ds(0, STRIPE)],
                    reg_sp.at[buf, pl.ds(0, STRIPE)], ssem).wait()

            @pl.when(sid == NUM_SUBCORES - 1)
            def _():
                off = 15 * STRIPE
                @pl.when(start + REG_ROWS <= n_board)
                def _():
                    pltpu.make_async_copy(
                        wb_hbm.at[pl.ds(0, REG_ROWS - off)],
                        reg_sp.at[buf, pl.ds(0, REG_ROWS - off)], ssem).wait()

                @pl.when(start + REG_ROWS > n_board)
                def _():
                    pltpu.make_async_copy(
                        wb_hbm.at[pl.ds(0, 192)],
                        reg_sp.at[buf, pl.ds(0, 192)], ssem).wait()

        def compute_cell(c, buf, next_c, next_buf, do_stage):
            _, skew = reg_start(c)
            reg = reg_sp.at[buf]

            pltpu.sync_copy(
                idx_hbm.at[pl.ds((c * NUM_SUBCORES + sid) * rec, rec)],
                raw_v)

            @pl.loop(0, bpt // LANES)
            def _(j):
                sl0 = pl.ds(j * LANES, LANES)
                sl1 = pl.ds(bpt + j * LANES, LANES)
                occ = (raw_v[pl.ds(2 * bpt + j * LANES, LANES)]
                       + raw_v[pl.ds(3 * bpt + j * LANES, LANES)]) > 0
                p0 = jnp.where(occ, PCODE_DIM, raw_v[sl0])
                p1 = jnp.where(occ, 2 * PCODE_DIM + 1,
                               raw_v[sl1] + (PCODE_DIM + 1))
                raw_v[sl0] = p0
                raw_v[sl1] = p1
                raw_v[pl.ds(2 * bpt + j * LANES, LANES)] = p0 + skew
                raw_v[pl.ds(3 * bpt + j * LANES, LANES)] = p1 + skew

            for g in range(bpt // SG):
                if g == 1 and do_stage is not None:
                    @pl.when(do_stage)
                    def _():
                        stage_region(next_c, next_buf)
                i0 = g * SG
                s0 = raw_v.at[pl.ds(i0, SG)]
                s1 = raw_v.at[pl.ds(bpt + i0, SG)]
                s2 = raw_v.at[pl.ds(2 * bpt + i0, SG)]
                s3 = raw_v.at[pl.ds(3 * bpt + i0, SG)]
                pltpu.async_copy(wp_sp.at[s0], g_v.at[0], gsem)
                pltpu.async_copy(wp_sp.at[s1], g_v.at[1], gsem)
                pltpu.async_copy(reg.at[s2], g_v.at[2], gsem)
                pltpu.async_copy(reg.at[s3], g_v.at[3], gsem)
                for j, tab in enumerate((wp_sp.at[s0], wp_sp.at[s1],
                                         reg.at[s2], reg.at[s3])):
                    pltpu.make_async_copy(tab, g_v.at[j], gsem).wait()

                @pl.loop(0, SG)
                def _(r):
                    for q in range(FEATURE_DIM // LANES):
                        sl = pl.ds(q * LANES, LANES)
                        g_v[3, r, sl] = (g_v[0, r, sl] + g_v[1, r, sl]) + (
                            g_v[2, r, sl] + g_v[3, r, sl])

                boff = pl.multiple_of(sid * bpt + i0, 8)
                pltpu.async_copy(
                    g_v.at[3], out_hbm.at[c, pl.ds(boff, SG)], osem)

            for g in range(bpt // SG):
                boff = pl.multiple_of(sid * bpt + g * SG, 8)
                pltpu.make_async_copy(
                    g_v.at[3], out_hbm.at[c, pl.ds(boff, SG)], osem).wait()

        plsc.subcore_barrier()

        stage_region(c_lo, 0)
        wait_region(c_lo, 0)
        plsc.subcore_barrier()

        n_pairs = n_cells // 2

        def do_pair(t, _):
            a = c_lo + 2 * t
            compute_cell(a, 0, a + 1, 1, do_stage=a + 1 < c_lo + n_cells)
            wait_region(a + 1, 1)
            plsc.subcore_barrier()
            more = a + 2 < c_lo + n_cells
            compute_cell(a + 1, 1, a + 2, 0, do_stage=more)

            @pl.when(more)
            def _():
                wait_region(a + 2, 0)

            plsc.subcore_barrier()
            return 0

        lax.fori_loop(0, n_pairs, do_pair, 0)

        @pl.when(n_cells % 2 == 1)
        def _():
            compute_cell(c_lo + n_cells - 1, 0, 0, 1, do_stage=None)

    return k(idx_in, w_pcode, w_board)


def _tc_transpose(s, batch):
    grp = 8

    def body(s_ref, o_ref):
        o_ref[...] = jnp.transpose(s_ref[...], (1, 2, 0))

    return pl.pallas_call(
        body,
        grid=(batch // grp,),
        in_specs=[
            pl.BlockSpec((CELL_DIM, grp, FEATURE_DIM), lambda i: (0, i, 0))
        ],
        out_specs=pl.BlockSpec((grp, FEATURE_DIM, CELL_DIM), lambda i: (i, 0, 0)),
        out_shape=jax.ShapeDtypeStruct((batch, FEATURE_DIM, CELL_DIM), jnp.float32),
    )(s)


def kernel(sparse_feature_input, sparse_feature_dim, board_input, W_pcode, W_board):
    del sparse_feature_dim
    batch = sparse_feature_input.shape[0]
    bpt = batch // NUM_SUBCORES
    pc = sparse_feature_input[:, 10:12].reshape(batch, 2, CELL_DIM)
    bd = board_input.reshape(batch, 2, CELL_DIM)
    raw = jnp.concatenate([pc, bd], axis=1)
    idx_in = jnp.transpose(raw, (2, 1, 0)).reshape(
        CELL_DIM, 4, NUM_SUBCORES, bpt)
    idx_in = jnp.transpose(idx_in, (0, 2, 1, 3)).reshape(-1)
    s = _sc_gather_sum(idx_in, W_pcode, W_board, batch)
    out = _tc_transpose(s, batch)
    return out.reshape(batch, FEATURE_DIM, BOARD_SIZE, BOARD_SIZE)


# speedup vs baseline: 2.1405x; 2.0540x over previous
import functools

import jax
import jax.numpy as jnp
from jax import lax
from jax.experimental import pallas as pl
from jax.experimental.pallas import tpu as pltpu
from jax.experimental.pallas import tpu_sc as plsc

FEATURE_DIM = 128
BOARD_SIZE = 15
PCODE_DIM = 2380
EMBED_DIM = 2 * (PCODE_DIM + 1)
CELL_DIM = BOARD_SIZE * BOARD_SIZE

NUM_CORES = 2
NUM_SUBCORES = 16
LANES = 16

STRIPE = 304
REG_ROWS = 4768
CELLS0 = 112
SG = 16


def _sc_gather_sum(idx_in, w_pcode, w_board, batch):
    bpt = batch // NUM_SUBCORES
    rec = 4 * bpt
    n_board = CELL_DIM * EMBED_DIM
    mesh = plsc.VectorSubcoreMesh(core_axis_name="c", subcore_axis_name="s")

    @functools.partial(
        pl.kernel,
        mesh=mesh,
        out_type=jax.ShapeDtypeStruct((CELL_DIM, batch, FEATURE_DIM),
                                      jnp.float32),
        scratch_types=[
            pltpu.VMEM_SHARED((EMBED_DIM, FEATURE_DIM), jnp.float32),
            pltpu.VMEM_SHARED((2, REG_ROWS, FEATURE_DIM), jnp.float32),
            pltpu.VMEM((rec,), jnp.int32),
            pltpu.VMEM((4, SG, FEATURE_DIM), jnp.float32),
            pltpu.SemaphoreType.DMA,
            pltpu.SemaphoreType.DMA,
            pltpu.SemaphoreType.DMA,
        ],
    )
    def k(idx_hbm, wp_hbm, wb_hbm, out_hbm, wp_sp, reg_sp, raw_v,
          g_v, gsem, osem, ssem):
        cid = lax.axis_index("c")
        sid = lax.axis_index("s")

        @pl.when(sid < NUM_SUBCORES - 1)
        def _():
            off = pl.multiple_of(sid * STRIPE, 8)
            pltpu.sync_copy(wp_hbm.at[pl.ds(off, STRIPE)],
                            wp_sp.at[pl.ds(off, STRIPE)])

        @pl.when(sid == NUM_SUBCORES - 1)
        def _():
            tail = EMBED_DIM - 15 * STRIPE
            pltpu.sync_copy(wp_hbm.at[pl.ds(15 * STRIPE, tail)],
                            wp_sp.at[pl.ds(15 * STRIPE, tail)])

        c_lo = cid * CELLS0
        n_cells = CELLS0 + cid

        def reg_start(c):
            src0 = c * EMBED_DIM
            start = pl.multiple_of(src0 // 8 * 8, 8)
            return start, src0 - start

        def stage_region(c, buf):
            start, _ = reg_start(c)

            @pl.when(sid < NUM_SUBCORES - 1)
            def _():
                off = pl.multiple_of(sid * STRIPE, 8)
                pltpu.async_copy(wb_hbm.at[pl.ds(start + off, STRIPE)],
                                 reg_sp.at[buf, pl.ds(off, STRIPE)], ssem)

            @pl.when(sid == NUM_SUBCORES - 1)
            def _():
                off = 15 * STRIPE
                @pl.when(start + REG_ROWS <= n_board)
                def _():
                    pltpu.async_copy(
                        wb_hbm.at[pl.ds(start + off, REG_ROWS - off)],
                        reg_sp.at[buf, pl.ds(off, REG_ROWS - off)], ssem)

                @pl.when(start + REG_ROWS > n_board)
                def _():
                    pltpu.async_copy(
                        wb_hbm.at[pl.ds(start + off, 192)],
                        reg_sp.at[buf, pl.ds(off, 192)], ssem)

                    @pl.loop(0, 10)
                    def _(i):
                        pltpu.sync_copy(
                            wb_hbm.at[pl.ds(start + off + 192 + i, 1)],
                            reg_sp.at[buf, pl.ds(off + 192 + i, 1)])

        def wait_region(c, buf):
            start, _ = reg_start(c)

            @pl.when(sid < NUM_SUBCORES - 1)
            def _():
                pltpu.make_async_copy(
                    wb_hbm.at[pl.ds(0, STRIPE)],
                    reg_sp.at[buf, pl.ds(0, STRIPE)], ssem).wait()

            @pl.when(sid == NUM_SUBCORES - 1)
            def _():
                off = 15 * STRIPE
                @pl.when(start + REG_ROWS <= n_board)
                def _():
                    pltpu.make_async_copy(
                        wb_hbm.at[pl.ds(0, REG_ROWS - off)],
                        reg_sp.at[buf, pl.ds(0, REG_ROWS - off)], ssem).wait()

                @pl.when(start + REG_ROWS > n_board)
                def _():
                    pltpu.make_async_copy(
                        wb_hbm.at[pl.ds(0, 192)],
                        reg_sp.at[buf, pl.ds(0, 192)], ssem).wait()

        def compute_cell(c, buf, next_c, next_buf, do_stage):
            _, skew = reg_start(c)
            reg = reg_sp.at[buf]

            pltpu.sync_copy(
                idx_hbm.at[pl.ds((c * NUM_SUBCORES + sid) * rec, rec)],
                raw_v)

            @pl.loop(0, bpt // LANES)
            def _(j):
                sl0 = pl.ds(j * LANES, LANES)
                sl1 = pl.ds(bpt + j * LANES, LANES)
                occ = (raw_v[pl.ds(2 * bpt + j * LANES, LANES)]
                       + raw_v[pl.ds(3 * bpt + j * LANES, LANES)]) > 0
                p0 = jnp.where(occ, PCODE_DIM, raw_v[sl0])
                p1 = jnp.where(occ, 2 * PCODE_DIM + 1,
                               raw_v[sl1] + (PCODE_DIM + 1))
                raw_v[sl0] = p0
                raw_v[sl1] = p1
                raw_v[pl.ds(2 * bpt + j * LANES, LANES)] = p0 + skew
                raw_v[pl.ds(3 * bpt + j * LANES, LANES)] = p1 + skew

            for g in range(bpt // SG):
                if g == 1 and do_stage is not None:
                    @pl.when(do_stage)
                    def _():
                        stage_region(next_c, next_buf)
                i0 = g * SG
                s0 = raw_v.at[pl.ds(i0, SG)]
                s1 = raw_v.at[pl.ds(bpt + i0, SG)]
                s2 = raw_v.at[pl.ds(2 * bpt + i0, SG)]
                s3 = raw_v.at[pl.ds(3 * bpt + i0, SG)]
                pltpu.async_copy(wp_sp.at[s0], g_v.at[0], gsem)
                pltpu.async_copy(wp_sp.at[s1], g_v.at[1], gsem)
                pltpu.async_copy(reg.at[s2], g_v.at[2], gsem)
                pltpu.async_copy(reg.at[s3], g_v.at[3], gsem)
                for j, tab in enumerate((wp_sp.at[s0], wp_sp.at[s1],
                                         reg.at[s2], reg.at[s3])):
                    pltpu.make_async_copy(tab, g_v.at[j], gsem).wait()

                @pl.loop(0, SG)
                def _(r):
                    for q in range(FEATURE_DIM // LANES):
                        sl = pl.ds(q * LANES, LANES)
                        g_v[3, r, sl] = (g_v[0, r, sl] + g_v[1, r, sl]) + (
                            g_v[2, r, sl] + g_v[3, r, sl])

                boff = pl.multiple_of(sid * bpt + i0, 8)
                pltpu.async_copy(
                    g_v.at[3], out_hbm.at[c, pl.ds(boff, SG)], osem)

            for g in range(bpt // SG):
                boff = pl.multiple_of(sid * bpt + g * SG, 8)
                pltpu.make_async_copy(
                    g_v.at[3], out_hbm.at[c, pl.ds(boff, SG)], osem).wait()

        plsc.subcore_barrier()

        stage_region(c_lo, 0)
        wait_region(c_lo, 0)
        plsc.subcore_barrier()

        n_pairs = n_cells // 2

        def do_pair(t, _):
            a = c_lo + 2 * t
            compute_cell(a, 0, a + 1, 1, do_stage=a + 1 < c_lo + n_cells)
            wait_region(a + 1, 1)
            plsc.subcore_barrier()
            more = a + 2 < c_lo + n_cells
            compute_cell(a + 1, 1, a + 2, 0, do_stage=more)

            @pl.when(more)
            def _():
                wait_region(a + 2, 0)

            plsc.subcore_barrier()
            return 0

        lax.fori_loop(0, n_pairs, do_pair, 0)

        @pl.when(n_cells % 2 == 1)
        def _():
            compute_cell(c_lo + n_cells - 1, 0, 0, 1, do_stage=None)

    return k(idx_in, w_pcode, w_board)


def _tc_transpose(s, batch):
    grp = 8

    def body(s_ref, o_ref):
        for g in range(grp):
            o_ref[g] = s_ref[:, g, :].T

    return pl.pallas_call(
        body,
        grid=(batch // grp,),
        in_specs=[
            pl.BlockSpec((CELL_DIM, grp, FEATURE_DIM), lambda i: (0, i, 0))
        ],
        out_specs=pl.BlockSpec((grp, FEATURE_DIM, CELL_DIM), lambda i: (i, 0, 0)),
        out_shape=jax.ShapeDtypeStruct((batch, FEATURE_DIM, CELL_DIM), jnp.float32),
    )(s)


def kernel(sparse_feature_input, sparse_feature_dim, board_input, W_pcode, W_board):
    del sparse_feature_dim
    batch = sparse_feature_input.shape[0]
    bpt = batch // NUM_SUBCORES
    pc = sparse_feature_input[:, 10:12].reshape(batch, 2, CELL_DIM)
    bd = board_input.reshape(batch, 2, CELL_DIM)
    raw = jnp.concatenate([pc, bd], axis=1)
    idx_in = jnp.transpose(raw, (2, 1, 0)).reshape(
        CELL_DIM, 4, NUM_SUBCORES, bpt)
    idx_in = jnp.transpose(idx_in, (0, 2, 1, 3)).reshape(-1)
    s = _sc_gather_sum(idx_in, W_pcode, W_board, batch)
    out = _tc_transpose(s, batch)
    return out.reshape(batch, FEATURE_DIM, BOARD_SIZE, BOARD_SIZE)


# speedup vs baseline: 2.2642x; 1.0578x over previous
import functools

import jax
import jax.numpy as jnp
from jax import lax
from jax.experimental import pallas as pl
from jax.experimental.pallas import tpu as pltpu
from jax.experimental.pallas import tpu_sc as plsc

FEATURE_DIM = 128
BOARD_SIZE = 15
PCODE_DIM = 2380
EMBED_DIM = 2 * (PCODE_DIM + 1)
CELL_DIM = BOARD_SIZE * BOARD_SIZE

NUM_CORES = 2
NUM_SUBCORES = 16
LANES = 16

STRIPE = 304
REG_ROWS = 4768
CELLS0 = 112
SG = 16


def _sc_gather_sum(idx_in, w_pcode, w_board, batch):
    bpt = batch // NUM_SUBCORES
    rec = 4 * bpt
    n_board = CELL_DIM * EMBED_DIM
    mesh = plsc.VectorSubcoreMesh(core_axis_name="c", subcore_axis_name="s")

    @functools.partial(
        pl.kernel,
        mesh=mesh,
        out_type=jax.ShapeDtypeStruct((CELL_DIM, batch, FEATURE_DIM),
                                      jnp.float32),
        scratch_types=[
            pltpu.VMEM_SHARED((EMBED_DIM, FEATURE_DIM), jnp.float32),
            pltpu.VMEM_SHARED((2, REG_ROWS, FEATURE_DIM), jnp.float32),
            pltpu.VMEM((2, rec), jnp.int32),
            pltpu.VMEM((4, SG, FEATURE_DIM), jnp.float32),
            pltpu.SemaphoreType.DMA,
            pltpu.SemaphoreType.DMA,
            pltpu.SemaphoreType.DMA,
            pltpu.SemaphoreType.DMA,
        ],
    )
    def k(idx_hbm, wp_hbm, wb_hbm, out_hbm, wp_sp, reg_sp, raw2_v,
          g_v, gsem, osem, ssem, rsem):
        cid = lax.axis_index("c")
        sid = lax.axis_index("s")

        @pl.when(sid < NUM_SUBCORES - 1)
        def _():
            off = pl.multiple_of(sid * STRIPE, 8)
            pltpu.sync_copy(wp_hbm.at[pl.ds(off, STRIPE)],
                            wp_sp.at[pl.ds(off, STRIPE)])

        @pl.when(sid == NUM_SUBCORES - 1)
        def _():
            tail = EMBED_DIM - 15 * STRIPE
            pltpu.sync_copy(wp_hbm.at[pl.ds(15 * STRIPE, tail)],
                            wp_sp.at[pl.ds(15 * STRIPE, tail)])

        c_lo = cid * CELLS0
        n_cells = CELLS0 + cid

        def reg_start(c):
            src0 = c * EMBED_DIM
            start = pl.multiple_of(src0 // 8 * 8, 8)
            return start, src0 - start

        def stage_region(c, buf):
            start, _ = reg_start(c)

            @pl.when(sid < NUM_SUBCORES - 1)
            def _():
                off = pl.multiple_of(sid * STRIPE, 8)
                pltpu.async_copy(wb_hbm.at[pl.ds(start + off, STRIPE)],
                                 reg_sp.at[buf, pl.ds(off, STRIPE)], ssem)

            @pl.when(sid == NUM_SUBCORES - 1)
            def _():
                off = 15 * STRIPE
                @pl.when(start + REG_ROWS <= n_board)
                def _():
                    pltpu.async_copy(
                        wb_hbm.at[pl.ds(start + off, REG_ROWS - off)],
                        reg_sp.at[buf, pl.ds(off, REG_ROWS - off)], ssem)

                @pl.when(start + REG_ROWS > n_board)
                def _():
                    pltpu.async_copy(
                        wb_hbm.at[pl.ds(start + off, 192)],
                        reg_sp.at[buf, pl.ds(off, 192)], ssem)

                    @pl.loop(0, 10)
                    def _(i):
                        pltpu.sync_copy(
                            wb_hbm.at[pl.ds(start + off + 192 + i, 1)],
                            reg_sp.at[buf, pl.ds(off + 192 + i, 1)])

        def wait_region(c, buf):
            start, _ = reg_start(c)

            @pl.when(sid < NUM_SUBCORES - 1)
            def _():
                pltpu.make_async_copy(
                    wb_hbm.at[pl.ds(0, STRIPE)],
                    reg_sp.at[buf, pl.ds(0, STRIPE)], ssem).wait()

            @pl.when(sid == NUM_SUBCORES - 1)
            def _():
                off = 15 * STRIPE
                @pl.when(start + REG_ROWS <= n_board)
                def _():
                    pltpu.make_async_copy(
                        wb_hbm.at[pl.ds(0, REG_ROWS - off)],
                        reg_sp.at[buf, pl.ds(0, REG_ROWS - off)], ssem).wait()

                @pl.when(start + REG_ROWS > n_board)
                def _():
                    pltpu.make_async_copy(
                        wb_hbm.at[pl.ds(0, 192)],
                        reg_sp.at[buf, pl.ds(0, 192)], ssem).wait()

        def stage_raw(c, buf):
            pltpu.async_copy(
                idx_hbm.at[pl.ds((c * NUM_SUBCORES + sid) * rec, rec)],
                raw2_v.at[buf], rsem)

        def wait_raw(buf):
            pltpu.make_async_copy(
                idx_hbm.at[pl.ds(0, rec)], raw2_v.at[buf], rsem).wait()

        def compute_cell(c, buf, next_c, next_buf, do_stage):
            _, skew = reg_start(c)
            reg = reg_sp.at[buf]
            raw_v = raw2_v.at[buf]

            wait_raw(buf)
            if do_stage is not None:
                @pl.when(do_stage)
                def _():
                    stage_raw(next_c, next_buf)

            @pl.loop(0, bpt // LANES)
            def _(j):
                sl0 = pl.ds(j * LANES, LANES)
                sl1 = pl.ds(bpt + j * LANES, LANES)
                occ = (raw_v[pl.ds(2 * bpt + j * LANES, LANES)]
                       + raw_v[pl.ds(3 * bpt + j * LANES, LANES)]) > 0
                p0 = jnp.where(occ, PCODE_DIM, raw_v[sl0])
                p1 = jnp.where(occ, 2 * PCODE_DIM + 1,
                               raw_v[sl1] + (PCODE_DIM + 1))
                raw_v[sl0] = p0
                raw_v[sl1] = p1
                raw_v[pl.ds(2 * bpt + j * LANES, LANES)] = p0 + skew
                raw_v[pl.ds(3 * bpt + j * LANES, LANES)] = p1 + skew

            for g in range(bpt // SG):
                if g == 1 and do_stage is not None:
                    @pl.when(do_stage)
                    def _():
                        stage_region(next_c, next_buf)
                i0 = g * SG
                s0 = raw_v.at[pl.ds(i0, SG)]
                s1 = raw_v.at[pl.ds(bpt + i0, SG)]
                s2 = raw_v.at[pl.ds(2 * bpt + i0, SG)]
                s3 = raw_v.at[pl.ds(3 * bpt + i0, SG)]
                pltpu.async_copy(wp_sp.at[s0], g_v.at[0], gsem)
                pltpu.async_copy(wp_sp.at[s1], g_v.at[1], gsem)
                pltpu.async_copy(reg.at[s2], g_v.at[2], gsem)
                pltpu.async_copy(reg.at[s3], g_v.at[3], gsem)
                for j, tab in enumerate((wp_sp.at[s0], wp_sp.at[s1],
                                         reg.at[s2], reg.at[s3])):
                    pltpu.make_async_copy(tab, g_v.at[j], gsem).wait()

                @pl.loop(0, SG)
                def _(r):
                    for q in range(FEATURE_DIM // LANES):
                        sl = pl.ds(q * LANES, LANES)
                        g_v[3, r, sl] = (g_v[0, r, sl] + g_v[1, r, sl]) + (
                            g_v[2, r, sl] + g_v[3, r, sl])

                boff = pl.multiple_of(sid * bpt + i0, 8)
                pltpu.async_copy(
                    g_v.at[3], out_hbm.at[c, pl.ds(boff, SG)], osem)

            for g in range(bpt // SG):
                boff = pl.multiple_of(sid * bpt + g * SG, 8)
                pltpu.make_async_copy(
                    g_v.at[3], out_hbm.at[c, pl.ds(boff, SG)], osem).wait()

        plsc.subcore_barrier()

        stage_region(c_lo, 0)
        stage_raw(c_lo, 0)
        wait_region(c_lo, 0)
        plsc.subcore_barrier()

        n_pairs = n_cells // 2

        def do_pair(t, _):
            a = c_lo + 2 * t
            compute_cell(a, 0, a + 1, 1, do_stage=a + 1 < c_lo + n_cells)
            wait_region(a + 1, 1)
            plsc.subcore_barrier()
            more = a + 2 < c_lo + n_cells
            compute_cell(a + 1, 1, a + 2, 0, do_stage=more)

            @pl.when(more)
            def _():
                wait_region(a + 2, 0)

            plsc.subcore_barrier()
            return 0

        lax.fori_loop(0, n_pairs, do_pair, 0)

        @pl.when(n_cells % 2 == 1)
        def _():
            compute_cell(c_lo + n_cells - 1, 0, 0, 1, do_stage=None)

    return k(idx_in, w_pcode, w_board)


def _tc_transpose(s, batch):
    grp = 8

    def body(s_ref, o_ref):
        for g in range(grp):
            o_ref[g] = s_ref[:, g, :].T

    return pl.pallas_call(
        body,
        grid=(batch // grp,),
        in_specs=[
            pl.BlockSpec((CELL_DIM, grp, FEATURE_DIM), lambda i: (0, i, 0))
        ],
        out_specs=pl.BlockSpec((grp, FEATURE_DIM, CELL_DIM), lambda i: (i, 0, 0)),
        out_shape=jax.ShapeDtypeStruct((batch, FEATURE_DIM, CELL_DIM), jnp.float32),
    )(s)


def kernel(sparse_feature_input, sparse_feature_dim, board_input, W_pcode, W_board):
    del sparse_feature_dim
    batch = sparse_feature_input.shape[0]
    bpt = batch // NUM_SUBCORES
    pc = sparse_feature_input[:, 10:12].reshape(batch, 2, CELL_DIM)
    bd = board_input.reshape(batch, 2, CELL_DIM)
    raw = jnp.concatenate([pc, bd], axis=1)
    idx_in = jnp.transpose(raw, (2, 1, 0)).reshape(
        CELL_DIM, 4, NUM_SUBCORES, bpt)
    idx_in = jnp.transpose(idx_in, (0, 2, 1, 3)).reshape(-1)
    s = _sc_gather_sum(idx_in, W_pcode, W_board, batch)
    out = _tc_transpose(s, batch)
    return out.reshape(batch, FEATURE_DIM, BOARD_SIZE, BOARD_SIZE)


# speedup vs baseline: 2.3720x; 1.0476x over previous
import functools

import jax
import jax.numpy as jnp
from jax import lax
from jax.experimental import pallas as pl
from jax.experimental.pallas import tpu as pltpu
from jax.experimental.pallas import tpu_sc as plsc

FEATURE_DIM = 128
BOARD_SIZE = 15
PCODE_DIM = 2380
EMBED_DIM = 2 * (PCODE_DIM + 1)
CELL_DIM = BOARD_SIZE * BOARD_SIZE

NUM_CORES = 2
NUM_SUBCORES = 16
LANES = 16

STRIPE = 304
REG_ROWS = 4768
CELLS0 = 112
SG = 16


def _sc_gather_sum(idx_in, w_pcode, w_board, batch):
    bpt = batch // NUM_SUBCORES
    rec = 4 * bpt
    n_board = CELL_DIM * EMBED_DIM
    mesh = plsc.VectorSubcoreMesh(core_axis_name="c", subcore_axis_name="s")

    @functools.partial(
        pl.kernel,
        mesh=mesh,
        out_type=jax.ShapeDtypeStruct((CELL_DIM, batch, FEATURE_DIM),
                                      jnp.float32),
        scratch_types=[
            pltpu.VMEM_SHARED((EMBED_DIM, FEATURE_DIM), jnp.float32),
            pltpu.VMEM_SHARED((2, REG_ROWS, FEATURE_DIM), jnp.float32),
            pltpu.VMEM((2, rec), jnp.int32),
            pltpu.VMEM((4, SG, FEATURE_DIM), jnp.float32),
            pltpu.SemaphoreType.DMA,
            pltpu.SemaphoreType.DMA,
            pltpu.SemaphoreType.DMA,
            pltpu.SemaphoreType.DMA,
        ],
    )
    def k(idx_hbm, wp_hbm, wb_hbm, out_hbm, wp_sp, reg_sp, raw2_v,
          g_v, gsem, osem, ssem, rsem):
        cid = lax.axis_index("c")
        sid = lax.axis_index("s")

        @pl.when(sid < NUM_SUBCORES - 1)
        def _():
            off = pl.multiple_of(sid * STRIPE, 8)
            pltpu.sync_copy(wp_hbm.at[pl.ds(off, STRIPE)],
                            wp_sp.at[pl.ds(off, STRIPE)])

        @pl.when(sid == NUM_SUBCORES - 1)
        def _():
            tail = EMBED_DIM - 15 * STRIPE
            pltpu.sync_copy(wp_hbm.at[pl.ds(15 * STRIPE, tail)],
                            wp_sp.at[pl.ds(15 * STRIPE, tail)])

        c_lo = cid * CELLS0
        n_cells = CELLS0 + cid

        def reg_start(c):
            src0 = c * EMBED_DIM
            start = pl.multiple_of(src0 // 8 * 8, 8)
            return start, src0 - start

        def stage_region(c, buf):
            start, _ = reg_start(c)

            @pl.when(sid < NUM_SUBCORES - 1)
            def _():
                off = pl.multiple_of(sid * STRIPE, 8)
                pltpu.async_copy(wb_hbm.at[pl.ds(start + off, STRIPE)],
                                 reg_sp.at[buf, pl.ds(off, STRIPE)], ssem)

            @pl.when(sid == NUM_SUBCORES - 1)
            def _():
                off = 15 * STRIPE
                @pl.when(start + REG_ROWS <= n_board)
                def _():
                    pltpu.async_copy(
                        wb_hbm.at[pl.ds(start + off, REG_ROWS - off)],
                        reg_sp.at[buf, pl.ds(off, REG_ROWS - off)], ssem)

                @pl.when(start + REG_ROWS > n_board)
                def _():
                    pltpu.async_copy(
                        wb_hbm.at[pl.ds(start + off, 192)],
                        reg_sp.at[buf, pl.ds(off, 192)], ssem)

                    @pl.loop(0, 10)
                    def _(i):
                        pltpu.sync_copy(
                            wb_hbm.at[pl.ds(start + off + 192 + i, 1)],
                            reg_sp.at[buf, pl.ds(off + 192 + i, 1)])

        def wait_region(c, buf):
            start, _ = reg_start(c)

            @pl.when(sid < NUM_SUBCORES - 1)
            def _():
                pltpu.make_async_copy(
                    wb_hbm.at[pl.ds(0, STRIPE)],
                    reg_sp.at[buf, pl.ds(0, STRIPE)], ssem).wait()

            @pl.when(sid == NUM_SUBCORES - 1)
            def _():
                off = 15 * STRIPE
                @pl.when(start + REG_ROWS <= n_board)
                def _():
                    pltpu.make_async_copy(
                        wb_hbm.at[pl.ds(0, REG_ROWS - off)],
                        reg_sp.at[buf, pl.ds(0, REG_ROWS - off)], ssem).wait()

                @pl.when(start + REG_ROWS > n_board)
                def _():
                    pltpu.make_async_copy(
                        wb_hbm.at[pl.ds(0, 192)],
                        reg_sp.at[buf, pl.ds(0, 192)], ssem).wait()

        def stage_raw(c, buf):
            pltpu.async_copy(
                idx_hbm.at[pl.ds((c * NUM_SUBCORES + sid) * rec, rec)],
                raw2_v.at[buf], rsem)

        def wait_raw(buf):
            pltpu.make_async_copy(
                idx_hbm.at[pl.ds(0, rec)], raw2_v.at[buf], rsem).wait()

        def compute_cell(c, buf, next_c, next_buf, do_stage):
            _, skew = reg_start(c)
            reg = reg_sp.at[buf]
            raw_v = raw2_v.at[buf]

            wait_raw(buf)
            if do_stage is not None:
                @pl.when(do_stage)
                def _():
                    stage_raw(next_c, next_buf)

            @pl.loop(0, bpt // LANES)
            def _(j):
                sl0 = pl.ds(j * LANES, LANES)
                sl1 = pl.ds(bpt + j * LANES, LANES)
                occ = (raw_v[pl.ds(2 * bpt + j * LANES, LANES)]
                       + raw_v[pl.ds(3 * bpt + j * LANES, LANES)]) > 0
                p0 = jnp.where(occ, PCODE_DIM, raw_v[sl0])
                p1 = jnp.where(occ, 2 * PCODE_DIM + 1,
                               raw_v[sl1] + (PCODE_DIM + 1))
                raw_v[sl0] = p0
                raw_v[sl1] = p1
                raw_v[pl.ds(2 * bpt + j * LANES, LANES)] = p0 + skew
                raw_v[pl.ds(3 * bpt + j * LANES, LANES)] = p1 + skew

            for g in range(bpt // SG):
                if g == 1 and do_stage is not None:
                    @pl.when(do_stage)
                    def _():
                        stage_region(next_c, next_buf)
                i0 = g * SG
                s0 = raw_v.at[pl.ds(i0, SG)]
                s1 = raw_v.at[pl.ds(bpt + i0, SG)]
                s2 = raw_v.at[pl.ds(2 * bpt + i0, SG)]
                s3 = raw_v.at[pl.ds(3 * bpt + i0, SG)]
                pltpu.async_copy(wp_sp.at[s0], g_v.at[0], gsem)
                pltpu.async_copy(wp_sp.at[s1], g_v.at[1], gsem)
                pltpu.async_copy(reg.at[s2], g_v.at[2], gsem)
                pltpu.async_copy(reg.at[s3], g_v.at[3], gsem)
                for j, tab in enumerate((wp_sp.at[s0], wp_sp.at[s1],
                                         reg.at[s2], reg.at[s3])):
                    pltpu.make_async_copy(tab, g_v.at[j], gsem).wait()

                @pl.loop(0, SG)
                def _(r):
                    for q in range(FEATURE_DIM // LANES):
                        sl = pl.ds(q * LANES, LANES)
                        g_v[3, r, sl] = (g_v[0, r, sl] + g_v[1, r, sl]) + (
                            g_v[2, r, sl] + g_v[3, r, sl])

                boff = pl.multiple_of(sid * bpt + i0, 8)
                pltpu.async_copy(
                    g_v.at[3], out_hbm.at[c, pl.ds(boff, SG)], osem)

            for g in range(bpt // SG):
                boff = pl.multiple_of(sid * bpt + g * SG, 8)
                pltpu.make_async_copy(
                    g_v.at[3], out_hbm.at[c, pl.ds(boff, SG)], osem).wait()

        plsc.subcore_barrier()

        stage_region(c_lo, 0)
        stage_raw(c_lo, 0)
        wait_region(c_lo, 0)
        plsc.subcore_barrier()

        n_pairs = n_cells // 2

        def do_pair(t, _):
            a = c_lo + 2 * t
            compute_cell(a, 0, a + 1, 1, do_stage=a + 1 < c_lo + n_cells)
            wait_region(a + 1, 1)
            plsc.subcore_barrier()
            more = a + 2 < c_lo + n_cells
            compute_cell(a + 1, 1, a + 2, 0, do_stage=more)

            @pl.when(more)
            def _():
                wait_region(a + 2, 0)

            plsc.subcore_barrier()
            return 0

        lax.fori_loop(0, n_pairs, do_pair, 0)

        @pl.when(n_cells % 2 == 1)
        def _():
            compute_cell(c_lo + n_cells - 1, 0, 0, 1, do_stage=None)

    return k(idx_in, w_pcode, w_board)


def _tc_transpose(s, batch):
    grp = 16

    def body(s_ref, o_ref):
        for g in range(grp):
            o_ref[g] = s_ref[:, g, :].T

    return pl.pallas_call(
        body,
        grid=(batch // grp,),
        in_specs=[
            pl.BlockSpec((CELL_DIM, grp, FEATURE_DIM), lambda i: (0, i, 0))
        ],
        out_specs=pl.BlockSpec((grp, FEATURE_DIM, CELL_DIM), lambda i: (i, 0, 0)),
        out_shape=jax.ShapeDtypeStruct((batch, FEATURE_DIM, CELL_DIM), jnp.float32),
    )(s)


def kernel(sparse_feature_input, sparse_feature_dim, board_input, W_pcode, W_board):
    del sparse_feature_dim
    batch = sparse_feature_input.shape[0]
    bpt = batch // NUM_SUBCORES
    pc = sparse_feature_input[:, 10:12].reshape(batch, 2, CELL_DIM)
    bd = board_input.reshape(batch, 2, CELL_DIM)
    raw = jnp.concatenate([pc, bd], axis=1)
    idx_in = jnp.transpose(raw, (2, 1, 0)).reshape(
        CELL_DIM, 4, NUM_SUBCORES, bpt)
    idx_in = jnp.transpose(idx_in, (0, 2, 1, 3)).reshape(-1)
    s = _sc_gather_sum(idx_in, W_pcode, W_board, batch)
    out = _tc_transpose(s, batch)
    return out.reshape(batch, FEATURE_DIM, BOARD_SIZE, BOARD_SIZE)
